# Initial kernel scaffold; baseline (speedup 1.0000x reference)
import jax
import jax.numpy as jnp
from jax.experimental import pallas as pl


def kernel(x, edge_index, edge_weight, W1, b1, W2, b2):
    raise NotImplementedError("write your pallas kernel here")


# speedup vs baseline: 7.4711x; 7.4711x over previous
import functools

import jax
import jax.numpy as jnp
from jax import lax
from jax.experimental import pallas as pl
from jax.experimental.pallas import tpu as pltpu
from jax.experimental.pallas import tpu_sc as plsc

N = 10000
NP = 10240
E = 320000
NW = 32
CH = 80
EP = NW * CH * 128
D1 = 128
DC = 40
D2 = 48

_mesh = plsc.VectorSubcoreMesh(core_axis_name="c", subcore_axis_name="s")
_sc_params = pltpu.CompilerParams(use_tc_tiling_on_sc=False)


@functools.partial(
    pl.kernel,
    mesh=_mesh,
    out_type=jax.ShapeDtypeStruct((2, NP), jnp.float32),
    scratch_types=[
        pltpu.VMEM((CH, 128), jnp.int32),
        pltpu.VMEM((CH, 128), jnp.float32),
        pltpu.VMEM((640,), jnp.float32),
        pltpu.VMEM_SHARED((NP,), jnp.float32),
    ],
    compiler_params=_sc_params,
)
def _sc_deg(dst_hbm, ew_hbm, out_hbm, dst_v, ew_v, zb, deg_sh):
    c = lax.axis_index("c")
    s = lax.axis_index("s")
    wid = s * 2 + c

    @pl.loop(0, 640, step=16)
    def _(i):
        zb[pl.ds(i, 16)] = jnp.zeros((16,), jnp.float32)

    pltpu.sync_copy(zb, deg_sh.at[pl.ds(s * 640, 640)])
    plsc.subcore_barrier()

    pltpu.sync_copy(dst_hbm.at[wid], dst_v)
    pltpu.sync_copy(ew_hbm.at[wid], ew_v)

    @pl.loop(0, CH)
    def _(k):
        pltpu.sync_copy(ew_v.at[k], deg_sh.at[dst_v.at[k]], add=True)

    plsc.subcore_barrier()
    pltpu.sync_copy(deg_sh.at[pl.ds(s * 640, 640)],
                    out_hbm.at[c, pl.ds(s * 640, 640)])


def _make_sc_agg(D, NH):
    nvr = D // 16

    @functools.partial(
        pl.kernel,
        mesh=_mesh,
        out_type=jax.ShapeDtypeStruct((2, NP, D), jnp.float32),
        scratch_types=[
            pltpu.VMEM((CH, 128), jnp.int32),
            pltpu.VMEM((CH, 128), jnp.int32),
            pltpu.VMEM((CH, 128), jnp.float32),
            pltpu.VMEM((128, D), jnp.float32),
            pltpu.VMEM((128, D), jnp.float32),
            pltpu.VMEM_SHARED((NP, D), jnp.float32),
        ],
        compiler_params=_sc_params,
    )
    def sc_agg(src_hbm, dst_hbm, ew_hbm, h_hbm, out_hbm,
               src_v, dst_v, ew_v, rows_v, zb, acc_sh):
        c = lax.axis_index("c")
        s = lax.axis_index("s")
        wid = s * 2 + c

        @pl.loop(0, 128)
        def _(r):
            for j in range(nvr):
                zb[r, pl.ds(16 * j, 16)] = jnp.zeros((16,), jnp.float32)

        @pl.loop(0, 5)
        def _(t):
            pltpu.sync_copy(zb, acc_sh.at[pl.ds(s * 640 + t * 128, 128)])

        plsc.subcore_barrier()

        pltpu.sync_copy(src_hbm.at[wid], src_v)
        pltpu.sync_copy(dst_hbm.at[wid], dst_v)
        pltpu.sync_copy(ew_hbm.at[wid], ew_v)

        @pl.loop(0, CH)
        def _(k):
            pltpu.sync_copy(h_hbm.at[src_v.at[k]], rows_v)

            @pl.loop(0, 128, step=16)
            def _(e0):
                ew16 = ew_v[k, pl.ds(e0, 16)]
                for i in range(16):
                    wv = jnp.full((16,), ew16[i], jnp.float32)
                    for j in range(nvr):
                        sl = pl.ds(16 * j, 16)
                        rows_v[e0 + i, sl] = rows_v[e0 + i, sl] * wv

            pltpu.sync_copy(rows_v, acc_sh.at[dst_v.at[k]], add=True)

        plsc.subcore_barrier()

        @pl.loop(0, 5)
        def _(t):
            pltpu.sync_copy(acc_sh.at[pl.ds(s * 640 + t * 128, 128)],
                            out_hbm.at[c, pl.ds(s * 640 + t * 128, 128)])

    return sc_agg


_sc_agg64 = _make_sc_agg(64, N)
_sc_agg2 = _make_sc_agg(D2, N)


def _tc_mm1_body(x_ref, w1_ref, h_ref):
    h_ref[...] = jnp.dot(x_ref[...], w1_ref[...],
                         preferred_element_type=jnp.float32)


def _tc_scale1_body(deg_ref, h_ref, dinv_ref, hpa_ref, hpb_ref):
    deg = deg_ref[0, :] + deg_ref[1, :] + 1.0
    dinv = lax.rsqrt(deg)
    dinv_ref[...] = dinv
    hp = h_ref[...] * dinv[:N, None]
    hpa_ref[...] = hp[:, :64]
    hpb_ref[...] = hp[:, 64:]


def _tc_mid_body(agga_ref, aggb_ref, hpa_ref, hpb_ref, dinv_ref, b1_ref,
                 w2_ref, hp2_ref):
    dinv = dinv_ref[...][:N, None]
    za = agga_ref[0, :N, :] + agga_ref[1, :N, :] + hpa_ref[...]
    zb = aggb_ref[0, :N, :] + aggb_ref[1, :N, :] + hpb_ref[...]
    z = jnp.concatenate([za, zb], axis=1)
    z = z * dinv + b1_ref[...][None, :]
    a = jnp.maximum(z, 0.0)
    h2 = jnp.dot(a, w2_ref[...], preferred_element_type=jnp.float32)
    hp2_ref[...] = h2 * dinv


def _tc_out_body(agg_ref, hp2_ref, dinv_ref, b2_ref, out_ref):
    z = (agg_ref[0, :N, :] + agg_ref[1, :N, :] + hp2_ref[...])
    z = z * dinv_ref[...][:N, None] + b2_ref[...][None, :]
    logits = z[:, :DC]
    m = jnp.max(logits, axis=1, keepdims=True)
    lse = jnp.log(jnp.sum(jnp.exp(logits - m), axis=1, keepdims=True)) + m
    out_ref[...] = logits - lse


def kernel(x, edge_index, edge_weight, W1, b1, W2, b2):
    src = edge_index[0].astype(jnp.int32)
    dst = edge_index[1].astype(jnp.int32)
    ew = edge_weight.astype(jnp.float32)

    pad = EP - E
    src_t = jnp.concatenate([src, jnp.zeros((pad,), jnp.int32)]).reshape(NW, CH, 128)
    dst_t = jnp.concatenate([dst, jnp.zeros((pad,), jnp.int32)]).reshape(NW, CH, 128)
    ew_t = jnp.concatenate([ew, jnp.zeros((pad,), jnp.float32)]).reshape(NW, CH, 128)

    deg_part = _sc_deg(dst_t, ew_t)

    h1 = pl.pallas_call(
        _tc_mm1_body,
        out_shape=jax.ShapeDtypeStruct((N, D1), jnp.float32),
    )(x, W1)

    dinv, hpa, hpb = pl.pallas_call(
        _tc_scale1_body,
        out_shape=(jax.ShapeDtypeStruct((NP,), jnp.float32),
                   jax.ShapeDtypeStruct((N, 64), jnp.float32),
                   jax.ShapeDtypeStruct((N, 64), jnp.float32)),
    )(deg_part, h1)

    agg1a = _sc_agg64(src_t, dst_t, ew_t, hpa)
    agg1b = _sc_agg64(src_t, dst_t, ew_t, hpb)

    W2p = jnp.zeros((D1, D2), jnp.float32).at[:, :DC].set(W2)
    b2p = jnp.zeros((D2,), jnp.float32).at[:DC].set(b2)

    hp2 = pl.pallas_call(
        _tc_mid_body,
        out_shape=jax.ShapeDtypeStruct((N, D2), jnp.float32),
    )(agg1a, agg1b, hpa, hpb, dinv, b1, W2p)

    agg2 = _sc_agg2(src_t, dst_t, ew_t, hp2)

    out = pl.pallas_call(
        _tc_out_body,
        out_shape=jax.ShapeDtypeStruct((N, DC), jnp.float32),
    )(agg2, hp2, dinv, b2p)

    return out


# speedup vs baseline: 11.1567x; 1.4933x over previous
import functools

import jax
import jax.numpy as jnp
from jax import lax
from jax.experimental import pallas as pl
from jax.experimental.pallas import tpu as pltpu
from jax.experimental.pallas import tpu_sc as plsc

N = 10000
NP = 10240
E = 320000
NW = 32
CH = 80
EP = NW * CH * 128
D1 = 128
DC = 40
D2 = 48

_mesh = plsc.VectorSubcoreMesh(core_axis_name="c", subcore_axis_name="s")
_sc_params = pltpu.CompilerParams(use_tc_tiling_on_sc=False)


@functools.partial(
    pl.kernel,
    mesh=_mesh,
    out_type=jax.ShapeDtypeStruct((2, NP), jnp.float32),
    scratch_types=[
        pltpu.VMEM((CH, 128), jnp.int32),
        pltpu.VMEM((CH, 128), jnp.float32),
        pltpu.VMEM((640,), jnp.float32),
        pltpu.VMEM_SHARED((NP,), jnp.float32),
    ],
    compiler_params=_sc_params,
)
def _sc_deg(dst_hbm, ew_hbm, out_hbm, dst_v, ew_v, zb, deg_sh):
    c = lax.axis_index("c")
    s = lax.axis_index("s")
    wid = s * 2 + c

    @pl.loop(0, 640, step=16)
    def _(i):
        zb[pl.ds(i, 16)] = jnp.zeros((16,), jnp.float32)

    pltpu.sync_copy(zb, deg_sh.at[pl.ds(s * 640, 640)])
    plsc.subcore_barrier()

    pltpu.sync_copy(dst_hbm.at[wid], dst_v)
    pltpu.sync_copy(ew_hbm.at[wid], ew_v)

    @pl.loop(0, CH)
    def _(k):
        pltpu.sync_copy(ew_v.at[k], deg_sh.at[dst_v.at[k]], add=True)

    plsc.subcore_barrier()
    pltpu.sync_copy(deg_sh.at[pl.ds(s * 640, 640)],
                    out_hbm.at[c, pl.ds(s * 640, 640)])


def _make_sc_agg(D, NH):
    nvr = D // 16
    NB = 4

    @functools.partial(
        pl.kernel,
        mesh=_mesh,
        out_type=jax.ShapeDtypeStruct((2, NP, D), jnp.float32),
        scratch_types=[
            pltpu.VMEM((CH, 128), jnp.int32),
            pltpu.VMEM((CH, 128), jnp.int32),
            pltpu.VMEM((CH, 128), jnp.float32),
            [pltpu.VMEM((128, D), jnp.float32) for _ in range(NB)],
            [pltpu.SemaphoreType.DMA for _ in range(NB)],
            [pltpu.SemaphoreType.DMA for _ in range(NB)],
            pltpu.VMEM_SHARED((NP, D), jnp.float32),
        ],
        compiler_params=_sc_params,
    )
    def sc_agg(src_hbm, dst_hbm, ew_hbm, h_hbm, out_hbm,
               src_v, dst_v, ew_v, bufs, gsems, ssems, acc_sh):
        c = lax.axis_index("c")
        s = lax.axis_index("s")
        wid = s * 2 + c

        @pl.loop(0, 128)
        def _(r):
            for j in range(nvr):
                bufs[0][r, pl.ds(16 * j, 16)] = jnp.zeros((16,), jnp.float32)

        @pl.loop(0, 5)
        def _(t):
            pltpu.sync_copy(bufs[0], acc_sh.at[pl.ds(s * 640 + t * 128, 128)])

        plsc.subcore_barrier()

        pltpu.sync_copy(src_hbm.at[wid], src_v)
        pltpu.sync_copy(dst_hbm.at[wid], dst_v)
        pltpu.sync_copy(ew_hbm.at[wid], ew_v)

        def issue_gather(k, p):
            pltpu.async_copy(h_hbm.at[src_v.at[k]], bufs[p], gsems[p])

        def wait_gather(k, p):
            pltpu.make_async_copy(h_hbm.at[src_v.at[k]], bufs[p],
                                  gsems[p]).wait()

        def issue_scatter(k, p):
            pltpu.async_copy(bufs[p], acc_sh.at[dst_v.at[k]], ssems[p],
                             add=True)

        def wait_scatter(k, p):
            pltpu.make_async_copy(bufs[p], acc_sh.at[dst_v.at[k]],
                                  ssems[p]).wait()

        def scale(k, p):
            buf = bufs[p]

            @pl.loop(0, 128, step=16)
            def _(e0):
                ew16 = ew_v[k, pl.ds(e0, 16)]
                for i in range(16):
                    wv = jnp.full((16,), ew16[i], jnp.float32)
                    for j in range(nvr):
                        sl = pl.ds(16 * j, 16)
                        buf[e0 + i, sl] = buf[e0 + i, sl] * wv

        def step(k, p, first, last):
            wait_gather(k, p)
            scale(k, p)
            if not first:
                wait_scatter(k - 1, (p + 3) % NB)
            issue_scatter(k, p)
            if not last:
                issue_gather(k + 2, (p + 2) % NB)

        issue_gather(0, 0)
        issue_gather(1, 1)
        step(0, 0, True, False)
        step(1, 1, False, False)

        @pl.loop(0, (CH - 4) // 4)
        def _(t):
            base = 4 * t + 2
            for i in range(4):
                step(base + i, (2 + i) % NB, False, False)

        step(CH - 2, (CH - 2) % NB, False, True)
        step(CH - 1, (CH - 1) % NB, False, True)
        wait_scatter(CH - 1, (CH - 1) % NB)

        plsc.subcore_barrier()

        @pl.loop(0, 5)
        def _(t):
            pltpu.sync_copy(acc_sh.at[pl.ds(s * 640 + t * 128, 128)],
                            out_hbm.at[c, pl.ds(s * 640 + t * 128, 128)])

    return sc_agg


_sc_agg64 = _make_sc_agg(64, N)
_sc_agg2 = _make_sc_agg(D2, N)


def _tc_mm1_body(x_ref, w1_ref, h_ref):
    h_ref[...] = jnp.dot(x_ref[...], w1_ref[...],
                         preferred_element_type=jnp.float32)


def _tc_scale1_body(deg_ref, h_ref, dinv_ref, hpa_ref, hpb_ref):
    deg = deg_ref[0, :] + deg_ref[1, :] + 1.0
    dinv = lax.rsqrt(deg)
    dinv_ref[...] = dinv
    hp = h_ref[...] * dinv[:N, None]
    hpa_ref[...] = hp[:, :64]
    hpb_ref[...] = hp[:, 64:]


def _tc_mid_body(agga_ref, aggb_ref, hpa_ref, hpb_ref, dinv_ref, b1_ref,
                 w2_ref, hp2_ref):
    dinv = dinv_ref[...][:N, None]
    za = agga_ref[0, :N, :] + agga_ref[1, :N, :] + hpa_ref[...]
    zb = aggb_ref[0, :N, :] + aggb_ref[1, :N, :] + hpb_ref[...]
    z = jnp.concatenate([za, zb], axis=1)
    z = z * dinv + b1_ref[...][None, :]
    a = jnp.maximum(z, 0.0)
    h2 = jnp.dot(a, w2_ref[...], preferred_element_type=jnp.float32)
    hp2_ref[...] = h2 * dinv


def _tc_out_body(agg_ref, hp2_ref, dinv_ref, b2_ref, out_ref):
    z = (agg_ref[0, :N, :] + agg_ref[1, :N, :] + hp2_ref[...])
    z = z * dinv_ref[...][:N, None] + b2_ref[...][None, :]
    logits = z[:, :DC]
    m = jnp.max(logits, axis=1, keepdims=True)
    lse = jnp.log(jnp.sum(jnp.exp(logits - m), axis=1, keepdims=True)) + m
    out_ref[...] = logits - lse


def kernel(x, edge_index, edge_weight, W1, b1, W2, b2):
    src = edge_index[0].astype(jnp.int32)
    dst = edge_index[1].astype(jnp.int32)
    ew = edge_weight.astype(jnp.float32)

    pad = EP - E
    src_t = jnp.concatenate([src, jnp.zeros((pad,), jnp.int32)]).reshape(NW, CH, 128)
    dst_t = jnp.concatenate([dst, jnp.zeros((pad,), jnp.int32)]).reshape(NW, CH, 128)
    ew_t = jnp.concatenate([ew, jnp.zeros((pad,), jnp.float32)]).reshape(NW, CH, 128)

    deg_part = _sc_deg(dst_t, ew_t)

    h1 = pl.pallas_call(
        _tc_mm1_body,
        out_shape=jax.ShapeDtypeStruct((N, D1), jnp.float32),
    )(x, W1)

    dinv, hpa, hpb = pl.pallas_call(
        _tc_scale1_body,
        out_shape=(jax.ShapeDtypeStruct((NP,), jnp.float32),
                   jax.ShapeDtypeStruct((N, 64), jnp.float32),
                   jax.ShapeDtypeStruct((N, 64), jnp.float32)),
    )(deg_part, h1)

    agg1a = _sc_agg64(src_t, dst_t, ew_t, hpa)
    agg1b = _sc_agg64(src_t, dst_t, ew_t, hpb)

    W2p = jnp.zeros((D1, D2), jnp.float32).at[:, :DC].set(W2)
    b2p = jnp.zeros((D2,), jnp.float32).at[:DC].set(b2)

    hp2 = pl.pallas_call(
        _tc_mid_body,
        out_shape=jax.ShapeDtypeStruct((N, D2), jnp.float32),
    )(agg1a, agg1b, hpa, hpb, dinv, b1, W2p)

    agg2 = _sc_agg2(src_t, dst_t, ew_t, hp2)

    out = pl.pallas_call(
        _tc_out_body,
        out_shape=jax.ShapeDtypeStruct((N, DC), jnp.float32),
    )(agg2, hp2, dinv, b2p)

    return out


# speedup vs baseline: 13.2454x; 1.1872x over previous
import functools

import jax
import jax.numpy as jnp
from jax import lax
from jax.experimental import pallas as pl
from jax.experimental.pallas import tpu as pltpu
from jax.experimental.pallas import tpu_sc as plsc

N = 10000
NP = 10240
E = 320000
CH0 = 96
CH1 = 64
CHM = max(CH0, CH1)
TCH = 16 * (CH0 + CH1)
EP = TCH * 128
D1 = 128
DC = 40
D2 = 48

_mesh = plsc.VectorSubcoreMesh(core_axis_name="c", subcore_axis_name="s")
_sc_params = pltpu.CompilerParams(use_tc_tiling_on_sc=False)


@functools.partial(
    pl.kernel,
    mesh=_mesh,
    out_type=jax.ShapeDtypeStruct((2, NP), jnp.float32),
    scratch_types=[
        pltpu.VMEM((CHM, 128), jnp.int32),
        pltpu.VMEM((CHM, 128), jnp.float32),
        pltpu.VMEM((640,), jnp.float32),
        pltpu.VMEM_SHARED((NP,), jnp.float32),
    ],
    compiler_params=_sc_params,
)
def _sc_deg(dst_hbm, ew_hbm, out_hbm, dst_v, ew_v, zb, deg_sh):
    c = lax.axis_index("c")
    s = lax.axis_index("s")
    ch = jnp.where(c == 0, CH0, CH1)

    @pl.loop(0, 640, step=16)
    def _(i):
        zb[pl.ds(i, 16)] = jnp.zeros((16,), jnp.float32)

    pltpu.sync_copy(zb, deg_sh.at[pl.ds(s * 640, 640)])
    plsc.subcore_barrier()

    pltpu.sync_copy(dst_hbm.at[c * 16 + s], dst_v)
    pltpu.sync_copy(ew_hbm.at[c * 16 + s], ew_v)

    @pl.loop(0, ch)
    def _(k):
        pltpu.sync_copy(ew_v.at[k], deg_sh.at[dst_v.at[k]], add=True)

    plsc.subcore_barrier()
    pltpu.sync_copy(deg_sh.at[pl.ds(s * 640, 640)],
                    out_hbm.at[c, pl.ds(s * 640, 640)])


def _make_sc_agg(D, NH):
    nvr = D // 16
    NB = 4

    @functools.partial(
        pl.kernel,
        mesh=_mesh,
        out_type=jax.ShapeDtypeStruct((2, NP, D), jnp.float32),
        scratch_types=[
            pltpu.VMEM((CHM, 128), jnp.int32),
            pltpu.VMEM((CHM, 128), jnp.int32),
            pltpu.VMEM((CHM, 128), jnp.float32),
            [pltpu.VMEM((128, D), jnp.float32) for _ in range(NB)],
            [pltpu.SemaphoreType.DMA for _ in range(NB)],
            [pltpu.SemaphoreType.DMA for _ in range(NB)],
            pltpu.VMEM_SHARED((NP, D), jnp.float32),
        ],
        compiler_params=_sc_params,
    )
    def sc_agg(src_hbm, dst_hbm, ew_hbm, h_hbm, out_hbm,
               src_v, dst_v, ew_v, bufs, gsems, ssems, acc_sh):
        c = lax.axis_index("c")
        s = lax.axis_index("s")
        ch = jnp.where(c == 0, CH0, CH1)

        @pl.loop(0, 128)
        def _(r):
            for j in range(nvr):
                bufs[0][r, pl.ds(16 * j, 16)] = jnp.zeros((16,), jnp.float32)

        @pl.loop(0, 5)
        def _(t):
            pltpu.sync_copy(bufs[0], acc_sh.at[pl.ds(s * 640 + t * 128, 128)])

        plsc.subcore_barrier()

        pltpu.sync_copy(src_hbm.at[c * 16 + s], src_v)
        pltpu.sync_copy(dst_hbm.at[c * 16 + s], dst_v)
        pltpu.sync_copy(ew_hbm.at[c * 16 + s], ew_v)

        def issue_gather(k, p):
            pltpu.async_copy(h_hbm.at[src_v.at[k]], bufs[p], gsems[p])

        def wait_gather(k, p):
            pltpu.make_async_copy(h_hbm.at[src_v.at[k]], bufs[p],
                                  gsems[p]).wait()

        def issue_scatter(k, p):
            pltpu.async_copy(bufs[p], acc_sh.at[dst_v.at[k]], ssems[p],
                             add=True)

        def wait_scatter(k, p):
            pltpu.make_async_copy(bufs[p], acc_sh.at[dst_v.at[k]],
                                  ssems[p]).wait()

        def scale(k, p):
            buf = bufs[p]

            @pl.loop(0, 128, step=16)
            def _(e0):
                ew16 = ew_v[k, pl.ds(e0, 16)]
                for i in range(16):
                    wv = jnp.full((16,), ew16[i], jnp.float32)
                    for j in range(nvr):
                        sl = pl.ds(16 * j, 16)
                        buf[e0 + i, sl] = buf[e0 + i, sl] * wv

        def step(k, p, first, last):
            wait_gather(k, p)
            scale(k, p)
            if not first:
                wait_scatter(k - 1, (p + 3) % NB)
            issue_scatter(k, p)
            if not last:
                issue_gather(k + 2, (p + 2) % NB)

        issue_gather(0, 0)
        issue_gather(1, 1)
        step(0, 0, True, False)
        step(1, 1, False, False)

        @pl.loop(0, (ch - 4) // 4)
        def _(t):
            base = 4 * t + 2
            for i in range(4):
                step(base + i, (2 + i) % NB, False, False)

        step(ch - 2, 2, False, True)
        step(ch - 1, 3, False, True)
        wait_scatter(ch - 1, 3)

        plsc.subcore_barrier()

        @pl.loop(0, 5)
        def _(t):
            pltpu.sync_copy(acc_sh.at[pl.ds(s * 640 + t * 128, 128)],
                            out_hbm.at[c, pl.ds(s * 640 + t * 128, 128)])

    return sc_agg


_sc_agg64 = _make_sc_agg(64, N)
_sc_agg2 = _make_sc_agg(D2, N)


def _tc_mm1_body(x_ref, w1_ref, h_ref):
    h_ref[...] = jnp.dot(x_ref[...], w1_ref[...],
                         preferred_element_type=jnp.float32)


def _tc_scale1_body(deg_ref, h_ref, dinv_ref, hpa_ref, hpb_ref):
    deg = deg_ref[0, :] + deg_ref[1, :] + 1.0
    dinv = lax.rsqrt(deg)
    dinv_ref[...] = dinv
    hp = h_ref[...] * dinv[:N, None]
    hpa_ref[...] = hp[:, :64]
    hpb_ref[...] = hp[:, 64:]


def _tc_mid_body(agga_ref, aggb_ref, hpa_ref, hpb_ref, dinv_ref, b1_ref,
                 w2_ref, hp2_ref):
    dinv = dinv_ref[...][:N, None]
    za = agga_ref[0, :N, :] + agga_ref[1, :N, :] + hpa_ref[...]
    zb = aggb_ref[0, :N, :] + aggb_ref[1, :N, :] + hpb_ref[...]
    z = jnp.concatenate([za, zb], axis=1)
    z = z * dinv + b1_ref[...][None, :]
    a = jnp.maximum(z, 0.0)
    h2 = jnp.dot(a, w2_ref[...], preferred_element_type=jnp.float32)
    hp2_ref[...] = h2 * dinv


def _tc_out_body(agg_ref, hp2_ref, dinv_ref, b2_ref, out_ref):
    z = (agg_ref[0, :N, :] + agg_ref[1, :N, :] + hp2_ref[...])
    z = z * dinv_ref[...][:N, None] + b2_ref[...][None, :]
    logits = z[:, :DC]
    m = jnp.max(logits, axis=1, keepdims=True)
    lse = jnp.log(jnp.sum(jnp.exp(logits - m), axis=1, keepdims=True)) + m
    out_ref[...] = logits - lse


def kernel(x, edge_index, edge_weight, W1, b1, W2, b2):
    src = edge_index[0].astype(jnp.int32)
    dst = edge_index[1].astype(jnp.int32)
    ew = edge_weight.astype(jnp.float32)

    def tile_edges(a):
        pad = EP - E
        flat = jnp.concatenate([a, jnp.zeros((pad,), a.dtype)]).reshape(TCH, 128)
        p0 = flat[:16 * CH0].reshape(16, CH0, 128)
        p1 = flat[16 * CH0:].reshape(16, CH1, 128)
        p1 = jnp.concatenate(
            [p1, jnp.zeros((16, CH0 - CH1, 128), a.dtype)], axis=1)
        return jnp.stack([p0, p1]).reshape(32, CHM, 128)

    src_t = tile_edges(src)
    dst_t = tile_edges(dst)
    ew_t = tile_edges(ew)

    deg_part = _sc_deg(dst_t, ew_t)

    h1 = pl.pallas_call(
        _tc_mm1_body,
        out_shape=jax.ShapeDtypeStruct((N, D1), jnp.float32),
    )(x, W1)

    dinv, hpa, hpb = pl.pallas_call(
        _tc_scale1_body,
        out_shape=(jax.ShapeDtypeStruct((NP,), jnp.float32),
                   jax.ShapeDtypeStruct((N, 64), jnp.float32),
                   jax.ShapeDtypeStruct((N, 64), jnp.float32)),
    )(deg_part, h1)

    agg1a = _sc_agg64(src_t, dst_t, ew_t, hpa)
    agg1b = _sc_agg64(src_t, dst_t, ew_t, hpb)

    W2p = jnp.zeros((D1, D2), jnp.float32).at[:, :DC].set(W2)
    b2p = jnp.zeros((D2,), jnp.float32).at[:DC].set(b2)

    hp2 = pl.pallas_call(
        _tc_mid_body,
        out_shape=jax.ShapeDtypeStruct((N, D2), jnp.float32),
    )(agg1a, agg1b, hpa, hpb, dinv, b1, W2p)

    agg2 = _sc_agg2(src_t, dst_t, ew_t, hp2)

    out = pl.pallas_call(
        _tc_out_body,
        out_shape=jax.ShapeDtypeStruct((N, DC), jnp.float32),
    )(agg2, hp2, dinv, b2p)

    return out
